# baseline (speedup 1.0000x reference)
import jax
import jax.numpy as jnp
import numpy as np
from jax.experimental import pallas as pl


def _spiral_map(cen):
    return {
        0: [(slice(1, 3), (cen - 1, slice(cen, cen + 2))),
            (slice(3, 5), (slice(cen, cen + 2), cen + 1)),
            (slice(5, 7), (cen + 1, slice(cen - 1, cen + 1))),
            (slice(7, 9), (slice(cen - 1, cen + 1), cen - 1))],
        1: [(slice(9, 13), (cen - 2, slice(cen - 1, cen + 3))),
            (slice(13, 17), (slice(cen - 1, cen + 3), cen + 2)),
            (slice(17, 21), (cen + 2, slice(cen - 2, cen + 2))),
            (slice(21, 25), (slice(cen - 2, cen + 2), cen - 2))],
        2: [(slice(25, 31), (cen - 3, slice(cen - 2, cen + 4))),
            (slice(31, 37), (slice(cen - 2, cen + 4), cen + 3)),
            (slice(37, 43), (cen + 3, slice(cen - 3, cen + 3))),
            (slice(43, 49), (slice(cen - 3, cen + 3), cen - 3))],
        3: [(slice(49, 57), (cen - 4, slice(cen - 3, cen + 5))),
            (slice(57, 65), (slice(cen - 3, cen + 5), cen + 4)),
            (slice(65, 73), (cen + 4, slice(cen - 4, cen + 4))),
            (slice(73, 81), (slice(cen - 4, cen + 4), cen - 4))],
        4: [(slice(81, 91), (cen - 5, slice(cen - 4, cen + 6))),
            (slice(91, 101), (slice(cen - 4, cen + 6), cen + 5)),
            (slice(101, 111), (cen + 5, slice(cen - 5, cen + 5))),
            (slice(111, 121), (slice(cen - 5, cen + 5), cen - 5))],
    }


def _src_perm(h):
    cen = h // 2
    src = np.empty(h * h, np.int64)
    src[0] = cen * h + cen
    for i in range(cen):
        for dest, (ri, ci) in _spiral_map(cen).get(i, []):
            if isinstance(ri, slice):
                cells = [(r, ci) for r in range(ri.start, ri.stop)]
            else:
                cells = [(ri, c) for c in range(ci.start, ci.stop)]
            for k, (r, c) in enumerate(cells):
                src[dest.start + k] = r * h + c
    return src


_H = 11
_HW = _H * _H

_SRC = _src_perm(_H)
_P3 = np.zeros((_H, _HW, _H), np.float32)
for _s in range(_HW):
    _r, _w = divmod(int(_SRC[_s]), _H)
    _P3[_r, _s, _w] = 1.0


def _body(p_ref, x_ref, o_ref):
    for i in range(x_ref.shape[0]):
        acc = None
        for r in range(_H):
            part = jax.lax.dot_general(
                p_ref[r], x_ref[i, :, r, :], (((1,), (1,)), ((), ())),
                preferred_element_type=jnp.float32)
            acc = part if acc is None else acc + part
        o_ref[i] = acc


def kernel(x):
    b, c, h, w = x.shape
    hw = h * w
    bb = 4
    assert b % bb == 0
    pmat = jnp.asarray(_P3)
    return pl.pallas_call(
        _body,
        grid=(b // bb,),
        in_specs=[
            pl.BlockSpec((h, hw, h), lambda i: (0, 0, 0)),
            pl.BlockSpec((bb, c, h, w), lambda i: (i, 0, 0, 0)),
        ],
        out_specs=pl.BlockSpec((bb, hw, c), lambda i: (i, 0, 0)),
        out_shape=jax.ShapeDtypeStruct((b, hw, c), x.dtype),
    )(pmat, x)


# speedup vs baseline: 3.0516x; 3.0516x over previous
import jax
import jax.numpy as jnp
import numpy as np
from jax.experimental import pallas as pl


def _spiral_map(cen):
    return {
        0: [(slice(1, 3), (cen - 1, slice(cen, cen + 2))),
            (slice(3, 5), (slice(cen, cen + 2), cen + 1)),
            (slice(5, 7), (cen + 1, slice(cen - 1, cen + 1))),
            (slice(7, 9), (slice(cen - 1, cen + 1), cen - 1))],
        1: [(slice(9, 13), (cen - 2, slice(cen - 1, cen + 3))),
            (slice(13, 17), (slice(cen - 1, cen + 3), cen + 2)),
            (slice(17, 21), (cen + 2, slice(cen - 2, cen + 2))),
            (slice(21, 25), (slice(cen - 2, cen + 2), cen - 2))],
        2: [(slice(25, 31), (cen - 3, slice(cen - 2, cen + 4))),
            (slice(31, 37), (slice(cen - 2, cen + 4), cen + 3)),
            (slice(37, 43), (cen + 3, slice(cen - 3, cen + 3))),
            (slice(43, 49), (slice(cen - 3, cen + 3), cen - 3))],
        3: [(slice(49, 57), (cen - 4, slice(cen - 3, cen + 5))),
            (slice(57, 65), (slice(cen - 3, cen + 5), cen + 4)),
            (slice(65, 73), (cen + 4, slice(cen - 4, cen + 4))),
            (slice(73, 81), (slice(cen - 4, cen + 4), cen - 4))],
        4: [(slice(81, 91), (cen - 5, slice(cen - 4, cen + 6))),
            (slice(91, 101), (slice(cen - 4, cen + 6), cen + 5)),
            (slice(101, 111), (cen + 5, slice(cen - 5, cen + 5))),
            (slice(111, 121), (slice(cen - 5, cen + 5), cen - 5))],
    }


def _src_perm(h):
    cen = h // 2
    src = np.empty(h * h, np.int64)
    src[0] = cen * h + cen
    for i in range(cen):
        for dest, (ri, ci) in _spiral_map(cen).get(i, []):
            if isinstance(ri, slice):
                cells = [(r, ci) for r in range(ri.start, ri.stop)]
            else:
                cells = [(ri, c) for c in range(ci.start, ci.stop)]
            for k, (r, c) in enumerate(cells):
                src[dest.start + k] = r * h + c
    return src


_H = 11
_HW = _H * _H
_PERM = np.zeros((_HW, _HW), np.float32)
_PERM[np.arange(_HW), _src_perm(_H)] = 1.0


def _body(p_ref, x_ref, o_ref):
    bb = x_ref.shape[0]
    c = x_ref.shape[1] // _HW
    xv = x_ref[...].reshape(bb, c, _HW)
    for i in range(bb):
        o_ref[i] = jax.lax.dot_general(
            p_ref[...], xv[i], (((1,), (1,)), ((), ())),
            preferred_element_type=jnp.float32)


def kernel(x):
    b, c, h, w = x.shape
    hw = h * w
    xr = x.reshape(b, c * hw)
    bb = 8
    assert b % bb == 0
    pmat = jnp.asarray(_PERM)
    return pl.pallas_call(
        _body,
        grid=(b // bb,),
        in_specs=[
            pl.BlockSpec((hw, hw), lambda i: (0, 0)),
            pl.BlockSpec((bb, c * hw), lambda i: (i, 0)),
        ],
        out_specs=pl.BlockSpec((bb, hw, c), lambda i: (i, 0, 0)),
        out_shape=jax.ShapeDtypeStruct((b, hw, c), x.dtype),
    )(pmat, xr)


# speedup vs baseline: 15.5594x; 5.0988x over previous
import jax
import jax.numpy as jnp
import numpy as np
from jax.experimental import pallas as pl
from jax.experimental.pallas import tpu as pltpu


def _spiral_map(cen):
    return {
        0: [(slice(1, 3), (cen - 1, slice(cen, cen + 2))),
            (slice(3, 5), (slice(cen, cen + 2), cen + 1)),
            (slice(5, 7), (cen + 1, slice(cen - 1, cen + 1))),
            (slice(7, 9), (slice(cen - 1, cen + 1), cen - 1))],
        1: [(slice(9, 13), (cen - 2, slice(cen - 1, cen + 3))),
            (slice(13, 17), (slice(cen - 1, cen + 3), cen + 2)),
            (slice(17, 21), (cen + 2, slice(cen - 2, cen + 2))),
            (slice(21, 25), (slice(cen - 2, cen + 2), cen - 2))],
        2: [(slice(25, 31), (cen - 3, slice(cen - 2, cen + 4))),
            (slice(31, 37), (slice(cen - 2, cen + 4), cen + 3)),
            (slice(37, 43), (cen + 3, slice(cen - 3, cen + 3))),
            (slice(43, 49), (slice(cen - 3, cen + 3), cen - 3))],
        3: [(slice(49, 57), (cen - 4, slice(cen - 3, cen + 5))),
            (slice(57, 65), (slice(cen - 3, cen + 5), cen + 4)),
            (slice(65, 73), (cen + 4, slice(cen - 4, cen + 4))),
            (slice(73, 81), (slice(cen - 4, cen + 4), cen - 4))],
        4: [(slice(81, 91), (cen - 5, slice(cen - 4, cen + 6))),
            (slice(91, 101), (slice(cen - 4, cen + 6), cen + 5)),
            (slice(101, 111), (cen + 5, slice(cen - 5, cen + 5))),
            (slice(111, 121), (slice(cen - 5, cen + 5), cen - 5))],
    }


def _src_perm(h):
    cen = h // 2
    src = np.empty(h * h, np.int64)
    src[0] = cen * h + cen
    for i in range(cen):
        for dest, (ri, ci) in _spiral_map(cen).get(i, []):
            if isinstance(ri, slice):
                cells = [(r, ci) for r in range(ri.start, ri.stop)]
            else:
                cells = [(ri, c) for c in range(ci.start, ci.stop)]
            for k, (r, c) in enumerate(cells):
                src[dest.start + k] = r * h + c
    return src


_H = 11
_HW = _H * _H


def _copy_body(src_ref, x_ref, o_ref):
    o_ref[...] = x_ref[...]


def kernel(x):
    b, c, h, w = x.shape
    hw = h * w
    xt = jnp.transpose(x, (2, 3, 0, 1)).reshape(hw, b, c)
    src = jnp.asarray(_src_perm(h), jnp.int32)
    grid_spec = pltpu.PrefetchScalarGridSpec(
        num_scalar_prefetch=1,
        grid=(hw,),
        in_specs=[pl.BlockSpec((1, b, c), lambda s, src_ref: (src_ref[s], 0, 0))],
        out_specs=pl.BlockSpec((1, b, c), lambda s, src_ref: (s, 0, 0)),
    )
    out_p = pl.pallas_call(
        _copy_body,
        grid_spec=grid_spec,
        out_shape=jax.ShapeDtypeStruct((hw, b, c), x.dtype),
    )(src, xt)
    return jnp.transpose(out_p, (1, 0, 2))
